# Initial kernel scaffold; baseline (speedup 1.0000x reference)
import functools

import jax
import jax.numpy as jnp
from jax import lax
from jax.experimental import pallas as pl
from jax.experimental.pallas import tpu as pltpu
from jax.experimental.pallas import tpu_sc as plsc

N_NODES = 10000
N_EDGES = 320000
D_IN = 128
D_HID = 128
N_CLS = 40
N_CLS_PAD = 48

NUM_CORES = 2
NUM_SUBCORES = 16
NUM_TILES = NUM_CORES * NUM_SUBCORES
EDGES_PER_TILE = N_EDGES // NUM_TILES
CHUNK = 80
NCHUNK = EDGES_PER_TILE // CHUNK
ROWS_PER_SUBCORE = N_NODES // NUM_SUBCORES


def _make_edge_agg(d: int):
  mesh = plsc.VectorSubcoreMesh(core_axis_name="c", subcore_axis_name="s")

  @functools.partial(
      pl.kernel,
      mesh=mesh,
      out_type=jax.ShapeDtypeStruct((NUM_CORES, N_NODES, d), jnp.float32),
      scratch_types=[
          pltpu.VMEM((NCHUNK, CHUNK), jnp.int32),
          pltpu.VMEM((NCHUNK, CHUNK), jnp.int32),
          pltpu.VMEM((CHUNK, d), jnp.float32),
          pltpu.VMEM_SHARED((N_NODES, d), jnp.float32),
          pltpu.SemaphoreType.DMA,
      ],
  )
  def agg(rows_hbm, src_hbm, dst_hbm, out_hbm, src_v, dst_v, rows_v, acc, sem):
    cid = lax.axis_index("c")
    sid = lax.axis_index("s")
    tid = cid * NUM_SUBCORES + sid

    pltpu.sync_copy(src_hbm.at[tid], src_v)
    pltpu.sync_copy(dst_hbm.at[tid], dst_v)

    def zero_body(i, _):
      r = i // (d // 16)
      c = i % (d // 16)
      rows_v[r, pl.ds(c * 16, 16)] = jnp.zeros((16,), jnp.float32)
      return 0
    lax.fori_loop(0, CHUNK * (d // 16), zero_body, 0)
    base = sid * ROWS_PER_SUBCORE
    for i in range(ROWS_PER_SUBCORE // CHUNK):
      pltpu.sync_copy(rows_v, acc.at[pl.ds(base + i * CHUNK, CHUNK)])
    rem = ROWS_PER_SUBCORE % CHUNK
    if rem:
      pltpu.sync_copy(
          rows_v.at[pl.ds(0, rem)],
          acc.at[pl.ds(base + ROWS_PER_SUBCORE - rem, rem)])
    plsc.subcore_barrier()

    def body(j, _):
      pltpu.async_copy(rows_hbm.at[src_v.at[j]], rows_v, sem).wait()
      pltpu.sync_copy(rows_v, acc.at[dst_v.at[j]], add=True)
      return 0
    lax.fori_loop(0, NCHUNK, body, 0)
    plsc.subcore_barrier()

    pltpu.sync_copy(acc.at[pl.ds(base, ROWS_PER_SUBCORE)],
                    out_hbm.at[cid, pl.ds(base, ROWS_PER_SUBCORE)])

  return agg


_agg_l1 = _make_edge_agg(D_IN)
_agg_l2 = _make_edge_agg(N_CLS_PAD)


def _mlp_body(p0, p1, w1, b1, w2, out):
  a = p0[0] + p1[0]
  h = lax.dot_general(a, w1[...], (((1,), (0,)), ((), ())),
                      preferred_element_type=jnp.float32)
  h = jnp.maximum(h + b1[...], 0.0)
  out[...] = lax.dot_general(h, w2[...], (((1,), (0,)), ((), ())),
                             preferred_element_type=jnp.float32)


def _logsoftmax_body(q0, q1, b2, out):
  s = q0[0] + q1[0] + b2[...]
  col = lax.broadcasted_iota(jnp.int32, s.shape, 1)
  s = jnp.where(col < N_CLS, s, -1e30)
  m = jnp.max(s, axis=1, keepdims=True)
  e = jnp.exp(s - m)
  den = jnp.sum(e, axis=1, keepdims=True)
  out[...] = (s - m - jnp.log(den))[:, :N_CLS]


_ROW_BLK = 1000


def kernel(x, edge_index, W1, b1, W2, b2):
  src = edge_index[0].reshape(NUM_TILES, NCHUNK, CHUNK)
  dst = edge_index[1].reshape(NUM_TILES, NCHUNK, CHUNK)
  W2p = jnp.pad(W2, ((0, 0), (0, N_CLS_PAD - N_CLS)))
  b2p = jnp.pad(b2, (0, N_CLS_PAD - N_CLS)).reshape(1, N_CLS_PAD)
  b1r = b1.reshape(1, D_HID)

  p = _agg_l1(x, src, dst)

  grid = (N_NODES // _ROW_BLK,)
  r = pl.pallas_call(
      _mlp_body,
      grid=grid,
      in_specs=[
          pl.BlockSpec((1, _ROW_BLK, D_IN), lambda i: (0, i, 0)),
          pl.BlockSpec((1, _ROW_BLK, D_IN), lambda i: (1, i, 0)),
          pl.BlockSpec((D_IN, D_HID), lambda i: (0, 0)),
          pl.BlockSpec((1, D_HID), lambda i: (0, 0)),
          pl.BlockSpec((D_HID, N_CLS_PAD), lambda i: (0, 0)),
      ],
      out_specs=pl.BlockSpec((_ROW_BLK, N_CLS_PAD), lambda i: (i, 0)),
      out_shape=jax.ShapeDtypeStruct((N_NODES, N_CLS_PAD), jnp.float32),
  )(p, p, W1, b1r, W2p)

  q = _agg_l2(r, src, dst)

  out = pl.pallas_call(
      _logsoftmax_body,
      grid=grid,
      in_specs=[
          pl.BlockSpec((1, _ROW_BLK, N_CLS_PAD), lambda i: (0, i, 0)),
          pl.BlockSpec((1, _ROW_BLK, N_CLS_PAD), lambda i: (1, i, 0)),
          pl.BlockSpec((1, N_CLS_PAD), lambda i: (0, 0)),
      ],
      out_specs=pl.BlockSpec((_ROW_BLK, N_CLS), lambda i: (i, 0)),
      out_shape=jax.ShapeDtypeStruct((N_NODES, N_CLS), jnp.float32),
  )(q, q, b2p)
  return out


# speedup vs baseline: 7.2154x; 7.2154x over previous
import functools

import jax
import jax.numpy as jnp
from jax import lax
from jax.experimental import pallas as pl
from jax.experimental.pallas import tpu as pltpu
from jax.experimental.pallas import tpu_sc as plsc

N_NODES = 10000
N_EDGES = 320000
D_IN = 128
D_HID = 128
N_CLS = 40
N_CLS_PAD = 128

NUM_CORES = 2
NUM_SUBCORES = 16
NUM_TILES = NUM_CORES * NUM_SUBCORES
EDGES_PER_TILE = N_EDGES // NUM_TILES
CHUNK = 80
NCHUNK = EDGES_PER_TILE // CHUNK
ROWS_PER_SUBCORE = 624
ROWS_TAIL = N_NODES - NUM_SUBCORES * ROWS_PER_SUBCORE


def _make_edge_agg(d: int):
  mesh = plsc.VectorSubcoreMesh(core_axis_name="c", subcore_axis_name="s")

  @functools.partial(
      pl.kernel,
      mesh=mesh,
      out_type=jax.ShapeDtypeStruct((NUM_CORES, N_NODES, d), jnp.float32),
      scratch_types=[
          pltpu.VMEM((NCHUNK, CHUNK), jnp.int32),
          pltpu.VMEM((NCHUNK, CHUNK), jnp.int32),
          pltpu.VMEM((CHUNK, d), jnp.float32),
          pltpu.VMEM_SHARED((N_NODES, d), jnp.float32),
          pltpu.SemaphoreType.DMA,
      ],
  )
  def agg(rows_hbm, src_hbm, dst_hbm, out_hbm, src_v, dst_v, rows_v, acc, sem):
    cid = lax.axis_index("c")
    sid = lax.axis_index("s")
    tid = cid * NUM_SUBCORES + sid

    pltpu.sync_copy(src_hbm.at[tid], src_v)
    pltpu.sync_copy(dst_hbm.at[tid], dst_v)

    def zero_body(i, _):
      r = i // (d // 16)
      c = i % (d // 16)
      rows_v[r, pl.ds(c * 16, 16)] = jnp.zeros((16,), jnp.float32)
      return 0
    lax.fori_loop(0, CHUNK * (d // 16), zero_body, 0)
    base = sid * ROWS_PER_SUBCORE
    for i in range(ROWS_PER_SUBCORE // CHUNK):
      pltpu.sync_copy(rows_v, acc.at[pl.ds(base + i * CHUNK, CHUNK)])
    rem = ROWS_PER_SUBCORE % CHUNK
    if rem:
      pltpu.sync_copy(
          rows_v.at[pl.ds(0, rem)],
          acc.at[pl.ds(base + ROWS_PER_SUBCORE - rem, rem)])

    @pl.when(sid == NUM_SUBCORES - 1)
    def _zero_tail():
      pltpu.sync_copy(
          rows_v.at[pl.ds(0, ROWS_TAIL)],
          acc.at[pl.ds(NUM_SUBCORES * ROWS_PER_SUBCORE, ROWS_TAIL)])
    plsc.subcore_barrier()

    def body(j, _):
      pltpu.async_copy(rows_hbm.at[src_v.at[j]], rows_v, sem).wait()
      pltpu.sync_copy(rows_v, acc.at[dst_v.at[j]], add=True)
      return 0
    lax.fori_loop(0, NCHUNK, body, 0)
    plsc.subcore_barrier()

    pltpu.sync_copy(acc.at[pl.ds(base, ROWS_PER_SUBCORE)],
                    out_hbm.at[cid, pl.ds(base, ROWS_PER_SUBCORE)])

    @pl.when(sid == NUM_SUBCORES - 1)
    def _out_tail():
      tb = NUM_SUBCORES * ROWS_PER_SUBCORE
      pltpu.sync_copy(acc.at[pl.ds(tb, ROWS_TAIL)],
                      out_hbm.at[cid, pl.ds(tb, ROWS_TAIL)])

  return agg


_agg_128 = _make_edge_agg(D_IN)


def _mlp_body(p0, p1, w1, b1, w2, out):
  a = p0[0] + p1[0]
  h = lax.dot_general(a, w1[...], (((1,), (0,)), ((), ())),
                      preferred_element_type=jnp.float32)
  h = jnp.maximum(h + b1[...], 0.0)
  out[...] = lax.dot_general(h, w2[...], (((1,), (0,)), ((), ())),
                             preferred_element_type=jnp.float32)


def _logsoftmax_body(q0, q1, b2, out):
  s = q0[0] + q1[0] + b2[...]
  col = lax.broadcasted_iota(jnp.int32, s.shape, 1)
  s = jnp.where(col < N_CLS, s, -1e30)
  m = jnp.max(s, axis=1, keepdims=True)
  e = jnp.exp(s - m)
  den = jnp.sum(e, axis=1, keepdims=True)
  out[...] = (s - m - jnp.log(den))[:, :N_CLS]


_ROW_BLK = 1000


def kernel(x, edge_index, W1, b1, W2, b2):
  src = edge_index[0].reshape(NUM_TILES, NCHUNK, CHUNK)
  dst = edge_index[1].reshape(NUM_TILES, NCHUNK, CHUNK)
  W2p = jnp.pad(W2, ((0, 0), (0, N_CLS_PAD - N_CLS)))
  b2p = jnp.pad(b2, (0, N_CLS_PAD - N_CLS)).reshape(1, N_CLS_PAD)
  b1r = b1.reshape(1, D_HID)

  p = _agg_128(x, src, dst)

  grid = (N_NODES // _ROW_BLK,)
  r = pl.pallas_call(
      _mlp_body,
      grid=grid,
      in_specs=[
          pl.BlockSpec((1, _ROW_BLK, D_IN), lambda i: (0, i, 0)),
          pl.BlockSpec((1, _ROW_BLK, D_IN), lambda i: (1, i, 0)),
          pl.BlockSpec((D_IN, D_HID), lambda i: (0, 0)),
          pl.BlockSpec((1, D_HID), lambda i: (0, 0)),
          pl.BlockSpec((D_HID, N_CLS_PAD), lambda i: (0, 0)),
      ],
      out_specs=pl.BlockSpec((_ROW_BLK, N_CLS_PAD), lambda i: (i, 0)),
      out_shape=jax.ShapeDtypeStruct((N_NODES, N_CLS_PAD), jnp.float32),
  )(p, p, W1, b1r, W2p)

  q = _agg_128(r, src, dst)

  out = pl.pallas_call(
      _logsoftmax_body,
      grid=grid,
      in_specs=[
          pl.BlockSpec((1, _ROW_BLK, N_CLS_PAD), lambda i: (0, i, 0)),
          pl.BlockSpec((1, _ROW_BLK, N_CLS_PAD), lambda i: (1, i, 0)),
          pl.BlockSpec((1, N_CLS_PAD), lambda i: (0, 0)),
      ],
      out_specs=pl.BlockSpec((_ROW_BLK, N_CLS), lambda i: (i, 0)),
      out_shape=jax.ShapeDtypeStruct((N_NODES, N_CLS), jnp.float32),
  )(q, q, b2p)
  return out
